# baseline (speedup 1.0000x reference)
import functools

import jax
import jax.numpy as jnp
from jax import lax
from jax.experimental import pallas as pl
from jax.experimental.pallas import tpu as pltpu
from jax.experimental.pallas import tpu_sc as plsc

_F = 64
_TAB_PAD = 256
_BM = 512
_CAP = 2048
_TCAP = 64


def _sc_sweep(table_t, tail_t, dest):
    B = dest.shape[0]
    V = table_t.shape[1]
    try:
        info = plsc.get_sparse_core_info()
        nc, ns = info.num_cores, info.num_subcores
    except Exception:
        nc, ns = 2, 16
    nw = nc * ns
    ng = V // 512
    baseq, rem = divmod(ng, nw)
    mesh = plsc.VectorSubcoreMesh(core_axis_name="c", subcore_axis_name="s")

    @functools.partial(
        pl.kernel,
        mesh=mesh,
        compiler_params=pltpu.CompilerParams(needs_layout_passes=False),
        out_type=jax.ShapeDtypeStruct((B + 16, 128), jnp.float32),
        scratch_types=[
            pltpu.VMEM((B,), jnp.int32),
            pltpu.VMEM((_CAP,), jnp.int32),
            pltpu.VMEM((_CAP,), jnp.int32),
            pltpu.VMEM((_TCAP,), jnp.int32),
            pltpu.VMEM((_TCAP,), jnp.int32),
            pltpu.VMEM((_F, 1024), jnp.float32),
            pltpu.VMEM((_F, V - ng * 512) if V % 512 else (_F, 128),
                       jnp.float32),
            pltpu.VMEM((32, 128), jnp.float32),
            pltpu.SemaphoreType.DMA,
            pltpu.SemaphoreType.DMA,
            pltpu.SemaphoreType.DMA,
            pltpu.SemaphoreType.DMA,
        ],
    )
    def sweep_k(dest_hbm, table_hbm, tail_hbm, out_hbm,
                dest_v, dlist, plist, md, mp, slab, tail_v, rows,
                sem_sl0, sem_sl1, sem_fl0, sem_fl1):
        wid = lax.axis_index("s") * nc + lax.axis_index("c")
        ntile = baseq + jnp.where(wid < rem, 1, 0)
        tstart = wid * baseq + jnp.minimum(wid, rem)
        c0 = tstart * 512
        is_last = wid == nw - 1
        cend = jnp.where(is_last, V, (tstart + ntile) * 512)
        iota16 = lax.broadcasted_iota(jnp.int32, (16,), 0)

        pltpu.sync_copy(dest_hbm, dest_v)
        pltpu.sync_copy(tail_hbm, tail_v)

        slab_sems = (sem_sl0, sem_sl1)

        def fire(t):
            ct = tstart + t
            h = lax.rem(t, 2)
            col0 = pl.multiple_of(ct * 512, 512)
            dsth = pl.multiple_of(h * 512, 512)
            for hh in range(2):
                @pl.when(h == hh)
                def _(hh=hh):
                    pltpu.make_async_copy(
                        table_hbm.at[:, pl.ds(col0, 512)],
                        slab.at[:, pl.ds(dsth, 512)], slab_sems[hh]).start()

        def wait_tile(t):
            h = lax.rem(t, 2)
            for hh in range(2):
                @pl.when(h == hh)
                def _(hh=hh):
                    pltpu.make_async_copy(
                        table_hbm.at[:, pl.ds(0, 512)],
                        slab.at[:, pl.ds(0, 512)], slab_sems[hh]).wait()

        def process(ct, h, carry, use_tail):

            def scan(j, cnt):
                j16 = pl.multiple_of(j * 16, 16)
                d = dlist[pl.ds(j16, 16)]
                p = plist[pl.ds(j16, 16)]
                m = ((iota16 + j * 16) < nitems) & (
                    lax.shift_right_logical(d, 9) == ct)
                s = plsc.cumsum(m.astype(jnp.int32))
                idx = jnp.minimum(cnt + s - 1, _TCAP - 1)
                plsc.store_scatter(md, [idx], d, mask=m)
                plsc.store_scatter(mp, [idx], p, mask=m)
                return cnt + s[15]
            cnt = lax.fori_loop(0, (nitems + 15) // 16, scan, 0)

            def ext(c, ec):
                slot, fpar, out_fl, pos16 = ec
                q16 = pl.multiple_of(lax.div(c, 16) * 16, 16)
                lane = jnp.full((16,), lax.rem(c, 16), jnp.int32)
                d16 = md[pl.ds(q16, 16)]
                p16 = mp[pl.ds(q16, 16)]
                r = d16.at[lane].get(mode="promise_in_bounds")
                pos = p16.at[lane].get(mode="promise_in_bounds")
                rowslot = jnp.full((16,), fpar * 16 + slot, jnp.int32)
                if use_tail:
                    lanecol = r & 127
                    src = tail_v
                else:
                    lanecol = h * 512 + (r & 511)
                    src = slab
                for k in range(4):
                    colk = plsc.load_gather(src, [iota16 + k * 16, lanecol])
                    plsc.store_scatter(rows, [rowslot, iota16 + k * 16], colk)
                pos16 = jnp.where(iota16 == slot, pos, pos16)

                def flush(args):
                    fpar, out_fl, pos16 = args
                    off = pl.multiple_of(fpar * 16, 16)

                    @pl.when(fpar == 0)
                    def _():
                        pltpu.make_async_copy(
                            rows.at[pl.ds(off, 16), :],
                            out_hbm.at[pos16], sem_fl0).start()

                    @pl.when(fpar == 1)
                    def _():
                        pltpu.make_async_copy(
                            rows.at[pl.ds(off, 16), :],
                            out_hbm.at[pos16], sem_fl1).start()

                    @pl.when((out_fl >= 1) & (fpar == 1))
                    def _():
                        pltpu.make_async_copy(
                            out_hbm.at[pl.ds(0, 16), :],
                            rows.at[pl.ds(0, 16), :], sem_fl0).wait()

                    @pl.when((out_fl >= 1) & (fpar == 0))
                    def _():
                        pltpu.make_async_copy(
                            out_hbm.at[pl.ds(0, 16), :],
                            rows.at[pl.ds(0, 16), :], sem_fl1).wait()
                    return (jnp.int32(0), jnp.int32(1) - fpar,
                            jnp.int32(1), B + iota16)

                def noflush(args):
                    fpar, out_fl, pos16 = args
                    return slot + jnp.int32(1), fpar, out_fl, pos16

                return lax.cond(slot + 1 == 16, flush, noflush,
                                (fpar, out_fl, pos16))

            return lax.fori_loop(0, cnt, ext, carry)

        fire(0)

        def p1(i, ptr):
            d = dest_v[pl.ds(pl.multiple_of(i * 16, 16), 16)]
            m = (d >= c0) & (d < cend)
            s = plsc.cumsum(m.astype(jnp.int32))
            idx = jnp.minimum(ptr + s - 1, _CAP - 1)
            plsc.store_scatter(dlist, [idx], d, mask=m)
            plsc.store_scatter(plist, [idx], iota16 + i * 16, mask=m)
            return ptr + s[15]
        nitems = jnp.minimum(lax.fori_loop(0, B // 16, p1, 0), _CAP)


        def tloop(t, carry):
            @pl.when(t + 1 < ntile)
            def _():
                fire(t + 1)
            wait_tile(t)
            return process(tstart + t, lax.rem(t, 2), carry, False)

        carry0 = (jnp.int32(0), jnp.int32(0), jnp.int32(0), B + iota16)
        carry1 = lax.fori_loop(0, ntile, tloop, carry0)
        slot, fpar, out_fl, pos16 = lax.cond(
            is_last,
            lambda c: process(jnp.int32(ng), jnp.int32(0), c, True),
            lambda c: c,
            carry1)

        off = pl.multiple_of(fpar * 16, 16)

        @pl.when(fpar == 0)
        def _():
            pltpu.make_async_copy(
                rows.at[pl.ds(off, 16), :], out_hbm.at[pos16], sem_fl0).start()
            pltpu.make_async_copy(
                out_hbm.at[pl.ds(0, 16), :],
                rows.at[pl.ds(0, 16), :], sem_fl0).wait()

        @pl.when(fpar == 1)
        def _():
            pltpu.make_async_copy(
                rows.at[pl.ds(off, 16), :], out_hbm.at[pos16], sem_fl1).start()
            pltpu.make_async_copy(
                out_hbm.at[pl.ds(0, 16), :],
                rows.at[pl.ds(0, 16), :], sem_fl1).wait()

        @pl.when((out_fl >= 1) & (fpar == 1))
        def _():
            pltpu.make_async_copy(
                out_hbm.at[pl.ds(0, 16), :],
                rows.at[pl.ds(0, 16), :], sem_fl0).wait()

        @pl.when((out_fl >= 1) & (fpar == 0))
        def _():
            pltpu.make_async_copy(
                out_hbm.at[pl.ds(0, 16), :],
                rows.at[pl.ds(0, 16), :], sem_fl1).wait()

    return sweep_k(dest, table_t, tail_t)


def _dense_body(idx_ref, cols_ref, tab_ref, w_ref, b_ref, out_ref):
    oh = jnp.zeros((_BM, _TAB_PAD), jnp.float32)
    iota = lax.broadcasted_iota(jnp.int32, (_BM, _TAB_PAD), 1)
    for k in range(6):
        oh = oh + (idx_ref[:, k : k + 1] == iota).astype(jnp.float32)
    user = jnp.dot(oh, tab_ref[...], preferred_element_type=jnp.float32,
                   precision=lax.Precision.DEFAULT)
    vec = user * cols_ref[:, :_F] * w_ref[0:1, :]
    s = jnp.sum(vec, axis=1, keepdims=True) + b_ref[0, 0]
    out_ref[...] = jnp.where(s >= 0, s, 0.01 * s)


def kernel(dayofweek, time, sex, age, month, day, destination,
           emb_dayofweek, emb_time, emb_sex, emb_age, emb_month, emb_day,
           item_table, W, b):
    B = destination.shape[0]
    dest = destination.astype(jnp.int32)

    table_t = item_table.T
    ntf = table_t.shape[1] // 512
    tail_t = table_t[:, ntf * 512 :]
    cols = _sc_sweep(table_t, tail_t, dest)

    tab = jnp.concatenate(
        [emb_dayofweek, emb_time, emb_sex, emb_age, emb_month, emb_day], axis=0)
    tab = jnp.pad(tab, ((0, _TAB_PAD - tab.shape[0]), (0, 0)))
    offs = (0, 7, 31, 33, 133, 145)
    feats = (dayofweek, time, sex, age, month, day)
    idx_cols = [f.astype(jnp.int32) + o for f, o in zip(feats, offs)]
    idx_cols += [jnp.zeros((B,), jnp.int32)] * 2
    idx_all = jnp.stack(idx_cols, axis=1)
    w_pad = jnp.pad(W, ((0, 7), (0, 0)))
    b2 = b.reshape(1, 1)

    nblk = B // _BM
    out = pl.pallas_call(
        _dense_body,
        grid=(nblk,),
        in_specs=[
            pl.BlockSpec((_BM, 8), lambda i: (i, 0)),
            pl.BlockSpec((_BM, 128), lambda i: (i, 0)),
            pl.BlockSpec((_TAB_PAD, _F), lambda i: (0, 0)),
            pl.BlockSpec((8, _F), lambda i: (0, 0)),
            pl.BlockSpec(memory_space=pltpu.SMEM),
        ],
        out_specs=pl.BlockSpec((_BM, 1), lambda i: (i, 0)),
        out_shape=jax.ShapeDtypeStruct((B, 1), jnp.float32),
    )(idx_all, cols, tab, w_pad, b2)
    return out.reshape(-1)


# speedup vs baseline: 1.0241x; 1.0241x over previous
import functools

import jax
import jax.numpy as jnp
from jax import lax
from jax.experimental import pallas as pl
from jax.experimental.pallas import tpu as pltpu
from jax.experimental.pallas import tpu_sc as plsc

_F = 64
_TAB_PAD = 256
_BM = 512
_CAP = 2048
_TCAP = 64


def _sc_sweep(table_t, tail_t, dest):
    B = dest.shape[0]
    V = table_t.shape[1]
    try:
        info = plsc.get_sparse_core_info()
        nc, ns = info.num_cores, info.num_subcores
    except Exception:
        nc, ns = 2, 16
    nw = nc * ns
    ng = V // 256
    baseq, rem = divmod(ng, nw)
    mesh = plsc.VectorSubcoreMesh(core_axis_name="c", subcore_axis_name="s")

    @functools.partial(
        pl.kernel,
        mesh=mesh,
        compiler_params=pltpu.CompilerParams(needs_layout_passes=False),
        out_type=jax.ShapeDtypeStruct((B + 16, 128), jnp.float32),
        scratch_types=[
            pltpu.VMEM((B,), jnp.int32),
            pltpu.VMEM((_CAP,), jnp.int32),
            pltpu.VMEM((_CAP,), jnp.int32),
            pltpu.VMEM((_TCAP,), jnp.int32),
            pltpu.VMEM((_TCAP,), jnp.int32),
            pltpu.VMEM((_F, 1024), jnp.float32),
            pltpu.VMEM((_F, V - ng * 256) if V % 256 else (_F, 128),
                       jnp.float32),
            pltpu.VMEM((32, 128), jnp.float32),
            pltpu.SemaphoreType.DMA,
            pltpu.SemaphoreType.DMA,
            pltpu.SemaphoreType.DMA,
            pltpu.SemaphoreType.DMA,
            pltpu.SemaphoreType.DMA,
            pltpu.SemaphoreType.DMA,
        ],
    )
    def sweep_k(dest_hbm, table_hbm, tail_hbm, out_hbm,
                dest_v, dlist, plist, md, mp, slab, tail_v, rows,
                sem_sl0, sem_sl1, sem_sl2, sem_sl3, sem_fl0, sem_fl1):
        wid = lax.axis_index("s") * nc + lax.axis_index("c")
        ntile = baseq + jnp.where(wid < rem, 1, 0)
        tstart = wid * baseq + jnp.minimum(wid, rem)
        c0 = tstart * 256
        is_last = wid == nw - 1
        cend = jnp.where(is_last, V, (tstart + ntile) * 256)
        iota16 = lax.broadcasted_iota(jnp.int32, (16,), 0)

        pltpu.sync_copy(dest_hbm, dest_v)
        pltpu.sync_copy(tail_hbm, tail_v)

        slab_sems = (sem_sl0, sem_sl1, sem_sl2, sem_sl3)

        def fire(t):
            ct = tstart + t
            h = lax.rem(t, 4)
            col0 = pl.multiple_of(ct * 256, 256)
            dsth = pl.multiple_of(h * 256, 256)
            for hh in range(4):
                @pl.when(h == hh)
                def _(hh=hh):
                    pltpu.make_async_copy(
                        table_hbm.at[:, pl.ds(col0, 256)],
                        slab.at[:, pl.ds(dsth, 256)], slab_sems[hh]).start()

        def wait_tile(t):
            h = lax.rem(t, 4)
            for hh in range(4):
                @pl.when(h == hh)
                def _(hh=hh):
                    pltpu.make_async_copy(
                        table_hbm.at[:, pl.ds(0, 256)],
                        slab.at[:, pl.ds(0, 256)], slab_sems[hh]).wait()

        def process(ct, h, carry, use_tail):

            def scan(j, cnt):
                j16 = pl.multiple_of(j * 16, 16)
                d = dlist[pl.ds(j16, 16)]
                p = plist[pl.ds(j16, 16)]
                m = ((iota16 + j * 16) < nitems) & (
                    lax.shift_right_logical(d, 8) == ct)
                s = plsc.cumsum(m.astype(jnp.int32))
                idx = jnp.minimum(cnt + s - 1, _TCAP - 1)
                plsc.store_scatter(md, [idx], d, mask=m)
                plsc.store_scatter(mp, [idx], p, mask=m)
                return cnt + s[15]
            cnt = lax.fori_loop(0, (nitems + 15) // 16, scan, 0)

            def ext(c, ec):
                slot, fpar, out_fl, pos16 = ec
                q16 = pl.multiple_of(lax.div(c, 16) * 16, 16)
                lane = jnp.full((16,), lax.rem(c, 16), jnp.int32)
                d16 = md[pl.ds(q16, 16)]
                p16 = mp[pl.ds(q16, 16)]
                r = d16.at[lane].get(mode="promise_in_bounds")
                pos = p16.at[lane].get(mode="promise_in_bounds")
                rowslot = jnp.full((16,), fpar * 16 + slot, jnp.int32)
                if use_tail:
                    lanecol = r & 127
                    src = tail_v
                else:
                    lanecol = h * 256 + (r & 255)
                    src = slab
                for k in range(4):
                    colk = plsc.load_gather(src, [iota16 + k * 16, lanecol])
                    plsc.store_scatter(rows, [rowslot, iota16 + k * 16], colk)
                pos16 = jnp.where(iota16 == slot, pos, pos16)

                def flush(args):
                    fpar, out_fl, pos16 = args
                    off = pl.multiple_of(fpar * 16, 16)

                    @pl.when(fpar == 0)
                    def _():
                        pltpu.make_async_copy(
                            rows.at[pl.ds(off, 16), :],
                            out_hbm.at[pos16], sem_fl0).start()

                    @pl.when(fpar == 1)
                    def _():
                        pltpu.make_async_copy(
                            rows.at[pl.ds(off, 16), :],
                            out_hbm.at[pos16], sem_fl1).start()

                    @pl.when((out_fl >= 1) & (fpar == 1))
                    def _():
                        pltpu.make_async_copy(
                            out_hbm.at[pl.ds(0, 16), :],
                            rows.at[pl.ds(0, 16), :], sem_fl0).wait()

                    @pl.when((out_fl >= 1) & (fpar == 0))
                    def _():
                        pltpu.make_async_copy(
                            out_hbm.at[pl.ds(0, 16), :],
                            rows.at[pl.ds(0, 16), :], sem_fl1).wait()
                    return (jnp.int32(0), jnp.int32(1) - fpar,
                            jnp.int32(1), B + iota16)

                def noflush(args):
                    fpar, out_fl, pos16 = args
                    return slot + jnp.int32(1), fpar, out_fl, pos16

                return lax.cond(slot + 1 == 16, flush, noflush,
                                (fpar, out_fl, pos16))

            return lax.fori_loop(0, cnt, ext, carry)

        fire(0)
        fire(1)
        fire(2)

        def p1(i, ptr):
            d = dest_v[pl.ds(pl.multiple_of(i * 16, 16), 16)]
            m = (d >= c0) & (d < cend)
            s = plsc.cumsum(m.astype(jnp.int32))
            idx = jnp.minimum(ptr + s - 1, _CAP - 1)
            plsc.store_scatter(dlist, [idx], d, mask=m)
            plsc.store_scatter(plist, [idx], iota16 + i * 16, mask=m)
            return ptr + s[15]
        nitems = jnp.minimum(lax.fori_loop(0, B // 16, p1, 0), _CAP)


        def tloop(t, carry):
            @pl.when(t + 3 < ntile)
            def _():
                fire(t + 3)
            wait_tile(t)
            return process(tstart + t, lax.rem(t, 4), carry, False)

        carry0 = (jnp.int32(0), jnp.int32(0), jnp.int32(0), B + iota16)
        carry1 = lax.fori_loop(0, ntile, tloop, carry0)
        slot, fpar, out_fl, pos16 = lax.cond(
            is_last,
            lambda c: process(jnp.int32(ng), jnp.int32(0), c, True),
            lambda c: c,
            carry1)

        off = pl.multiple_of(fpar * 16, 16)

        @pl.when(fpar == 0)
        def _():
            pltpu.make_async_copy(
                rows.at[pl.ds(off, 16), :], out_hbm.at[pos16], sem_fl0).start()
            pltpu.make_async_copy(
                out_hbm.at[pl.ds(0, 16), :],
                rows.at[pl.ds(0, 16), :], sem_fl0).wait()

        @pl.when(fpar == 1)
        def _():
            pltpu.make_async_copy(
                rows.at[pl.ds(off, 16), :], out_hbm.at[pos16], sem_fl1).start()
            pltpu.make_async_copy(
                out_hbm.at[pl.ds(0, 16), :],
                rows.at[pl.ds(0, 16), :], sem_fl1).wait()

        @pl.when((out_fl >= 1) & (fpar == 1))
        def _():
            pltpu.make_async_copy(
                out_hbm.at[pl.ds(0, 16), :],
                rows.at[pl.ds(0, 16), :], sem_fl0).wait()

        @pl.when((out_fl >= 1) & (fpar == 0))
        def _():
            pltpu.make_async_copy(
                out_hbm.at[pl.ds(0, 16), :],
                rows.at[pl.ds(0, 16), :], sem_fl1).wait()

    return sweep_k(dest, table_t, tail_t)


def _dense_body(idx_ref, cols_ref, tab_ref, w_ref, b_ref, out_ref):
    oh = jnp.zeros((_BM, _TAB_PAD), jnp.float32)
    iota = lax.broadcasted_iota(jnp.int32, (_BM, _TAB_PAD), 1)
    for k in range(6):
        oh = oh + (idx_ref[:, k : k + 1] == iota).astype(jnp.float32)
    user = jnp.dot(oh, tab_ref[...], preferred_element_type=jnp.float32,
                   precision=lax.Precision.DEFAULT)
    vec = user * cols_ref[:, :_F] * w_ref[0:1, :]
    s = jnp.sum(vec, axis=1, keepdims=True) + b_ref[0, 0]
    out_ref[...] = jnp.where(s >= 0, s, 0.01 * s)


def kernel(dayofweek, time, sex, age, month, day, destination,
           emb_dayofweek, emb_time, emb_sex, emb_age, emb_month, emb_day,
           item_table, W, b):
    B = destination.shape[0]
    dest = destination.astype(jnp.int32)

    table_t = item_table.T
    ntf = table_t.shape[1] // 256
    tail_t = table_t[:, ntf * 256 :]
    cols = _sc_sweep(table_t, tail_t, dest)

    tab = jnp.concatenate(
        [emb_dayofweek, emb_time, emb_sex, emb_age, emb_month, emb_day], axis=0)
    tab = jnp.pad(tab, ((0, _TAB_PAD - tab.shape[0]), (0, 0)))
    offs = (0, 7, 31, 33, 133, 145)
    feats = (dayofweek, time, sex, age, month, day)
    idx_cols = [f.astype(jnp.int32) + o for f, o in zip(feats, offs)]
    idx_cols += [jnp.zeros((B,), jnp.int32)] * 2
    idx_all = jnp.stack(idx_cols, axis=1)
    w_pad = jnp.pad(W, ((0, 7), (0, 0)))
    b2 = b.reshape(1, 1)

    nblk = B // _BM
    out = pl.pallas_call(
        _dense_body,
        grid=(nblk,),
        in_specs=[
            pl.BlockSpec((_BM, 8), lambda i: (i, 0)),
            pl.BlockSpec((_BM, 128), lambda i: (i, 0)),
            pl.BlockSpec((_TAB_PAD, _F), lambda i: (0, 0)),
            pl.BlockSpec((8, _F), lambda i: (0, 0)),
            pl.BlockSpec(memory_space=pltpu.SMEM),
        ],
        out_specs=pl.BlockSpec((_BM, 1), lambda i: (i, 0)),
        out_shape=jax.ShapeDtypeStruct((B, 1), jnp.float32),
    )(idx_all, cols, tab, w_pad, b2)
    return out.reshape(-1)


# speedup vs baseline: 1.0273x; 1.0031x over previous
import functools

import jax
import jax.numpy as jnp
from jax import lax
from jax.experimental import pallas as pl
from jax.experimental.pallas import tpu as pltpu
from jax.experimental.pallas import tpu_sc as plsc

_F = 64
_TAB_PAD = 256
_BM = 512
_CAP = 2048
_TCAP = 64


def _sc_sweep(table_t, tail_t, dest):
    B = dest.shape[0]
    V = table_t.shape[1]
    try:
        info = plsc.get_sparse_core_info()
        nc, ns = info.num_cores, info.num_subcores
    except Exception:
        nc, ns = 2, 16
    nw = nc * ns
    ng = V // 256
    baseq, rem = divmod(ng, nw)
    mesh = plsc.VectorSubcoreMesh(core_axis_name="c", subcore_axis_name="s")

    @functools.partial(
        pl.kernel,
        mesh=mesh,
        compiler_params=pltpu.CompilerParams(needs_layout_passes=False),
        out_type=jax.ShapeDtypeStruct((B + 16, 128), jnp.float32),
        scratch_types=[
            pltpu.VMEM((B,), jnp.int32),
            pltpu.VMEM((_CAP,), jnp.int32),
            pltpu.VMEM((_CAP,), jnp.int32),
            pltpu.VMEM((_TCAP,), jnp.int32),
            pltpu.VMEM((_TCAP,), jnp.int32),
            pltpu.VMEM((_F, 1024), jnp.float32),
            pltpu.VMEM((_F, V - ng * 256) if V % 256 else (_F, 128),
                       jnp.float32),
            pltpu.VMEM((32, 128), jnp.float32),
            pltpu.SemaphoreType.DMA,
            pltpu.SemaphoreType.DMA,
            pltpu.SemaphoreType.DMA,
            pltpu.SemaphoreType.DMA,
            pltpu.SemaphoreType.DMA,
            pltpu.SemaphoreType.DMA,
        ],
    )
    def sweep_k(dest_hbm, table_hbm, tail_hbm, out_hbm,
                dest_v, dlist, plist, md, mp, slab, tail_v, rows,
                sem_sl0, sem_sl1, sem_sl2, sem_sl3, sem_fl0, sem_fl1):
        wid = lax.axis_index("s") * nc + lax.axis_index("c")
        ntile = baseq + jnp.where(wid < rem, 1, 0)
        tstart = wid * baseq + jnp.minimum(wid, rem)
        c0 = tstart * 256
        is_last = wid == nw - 1
        cend = jnp.where(is_last, V, (tstart + ntile) * 256)
        iota16 = lax.broadcasted_iota(jnp.int32, (16,), 0)

        pltpu.sync_copy(dest_hbm, dest_v)
        pltpu.sync_copy(tail_hbm, tail_v)

        slab_sems = (sem_sl0, sem_sl1, sem_sl2, sem_sl3)

        def fire(t):
            ct = tstart + t
            h = lax.rem(t, 4)
            col0 = pl.multiple_of(ct * 256, 256)
            dsth = pl.multiple_of(h * 256, 256)
            for hh in range(4):
                @pl.when(h == hh)
                def _(hh=hh):
                    pltpu.make_async_copy(
                        table_hbm.at[:, pl.ds(col0, 256)],
                        slab.at[:, pl.ds(dsth, 256)], slab_sems[hh]).start()

        def wait_tile(t):
            h = lax.rem(t, 4)
            for hh in range(4):
                @pl.when(h == hh)
                def _(hh=hh):
                    pltpu.make_async_copy(
                        table_hbm.at[:, pl.ds(0, 256)],
                        slab.at[:, pl.ds(0, 256)], slab_sems[hh]).wait()

        def process(ct, h, carry, use_tail):

            def scan(j, cnt):
                j16 = pl.multiple_of(j * 16, 16)
                d = dlist[pl.ds(j16, 16)]
                p = plist[pl.ds(j16, 16)]
                m = ((iota16 + j * 16) < nitems) & (
                    lax.shift_right_logical(d, 8) == ct)
                s = plsc.cumsum(m.astype(jnp.int32))
                idx = jnp.minimum(cnt + s - 1, _TCAP - 1)
                plsc.store_scatter(md, [idx], d, mask=m)
                plsc.store_scatter(mp, [idx], p, mask=m)
                return cnt + s[15]
            cnt = lax.fori_loop(0, (nitems + 15) // 16, scan, 0)

            def ext(c, ec):
                slot, fpar, out_fl, pos16 = ec
                q16 = pl.multiple_of(lax.div(c, 16) * 16, 16)
                lane = jnp.full((16,), lax.rem(c, 16), jnp.int32)
                d16 = md[pl.ds(q16, 16)]
                p16 = mp[pl.ds(q16, 16)]
                r = d16.at[lane].get(mode="promise_in_bounds")
                pos = p16.at[lane].get(mode="promise_in_bounds")
                rowslot = jnp.full((16,), fpar * 16 + slot, jnp.int32)
                if use_tail:
                    lanecol = r & 127
                    src = tail_v
                else:
                    lanecol = h * 256 + (r & 255)
                    src = slab
                for k in range(4):
                    colk = plsc.load_gather(src, [iota16 + k * 16, lanecol])
                    plsc.store_scatter(rows, [rowslot, iota16 + k * 16], colk)
                pos16 = jnp.where(iota16 == slot, pos, pos16)

                def flush(args):
                    fpar, out_fl, pos16 = args
                    off = pl.multiple_of(fpar * 16, 16)

                    @pl.when(fpar == 0)
                    def _():
                        pltpu.make_async_copy(
                            rows.at[pl.ds(off, 16), :],
                            out_hbm.at[pos16], sem_fl0).start()

                    @pl.when(fpar == 1)
                    def _():
                        pltpu.make_async_copy(
                            rows.at[pl.ds(off, 16), :],
                            out_hbm.at[pos16], sem_fl1).start()

                    @pl.when((out_fl >= 1) & (fpar == 1))
                    def _():
                        pltpu.make_async_copy(
                            out_hbm.at[pl.ds(0, 16), :],
                            rows.at[pl.ds(0, 16), :], sem_fl0).wait()

                    @pl.when((out_fl >= 1) & (fpar == 0))
                    def _():
                        pltpu.make_async_copy(
                            out_hbm.at[pl.ds(0, 16), :],
                            rows.at[pl.ds(0, 16), :], sem_fl1).wait()
                    return (jnp.int32(0), jnp.int32(1) - fpar,
                            jnp.int32(1), B + iota16)

                def noflush(args):
                    fpar, out_fl, pos16 = args
                    return slot + jnp.int32(1), fpar, out_fl, pos16

                return lax.cond(slot + 1 == 16, flush, noflush,
                                (fpar, out_fl, pos16))

            return lax.fori_loop(0, cnt, ext, carry)

        fire(0)
        fire(1)
        fire(2)

        def p1(i, ptr):
            d = dest_v[pl.ds(pl.multiple_of(i * 16, 16), 16)]
            m = (d >= c0) & (d < cend)
            s = plsc.cumsum(m.astype(jnp.int32))
            idx = jnp.minimum(ptr + s - 1, _CAP - 1)
            plsc.store_scatter(dlist, [idx], d, mask=m)
            plsc.store_scatter(plist, [idx], iota16 + i * 16, mask=m)
            return ptr + s[15]
        nitems = jnp.minimum(lax.fori_loop(0, B // 16, p1, 0), _CAP)


        def tloop(t, carry):
            @pl.when(t + 3 < ntile)
            def _():
                fire(t + 3)
            wait_tile(t)
            return process(tstart + t, lax.rem(t, 4), carry, False)

        carry0 = (jnp.int32(0), jnp.int32(0), jnp.int32(0), B + iota16)
        carry1 = lax.fori_loop(0, ntile, tloop, carry0)
        slot, fpar, out_fl, pos16 = lax.cond(
            is_last,
            lambda c: process(jnp.int32(ng), jnp.int32(0), c, True),
            lambda c: c,
            carry1)

        off = pl.multiple_of(fpar * 16, 16)

        @pl.when(fpar == 0)
        def _():
            pltpu.make_async_copy(
                rows.at[pl.ds(off, 16), :], out_hbm.at[pos16], sem_fl0).start()
            pltpu.make_async_copy(
                out_hbm.at[pl.ds(0, 16), :],
                rows.at[pl.ds(0, 16), :], sem_fl0).wait()

        @pl.when(fpar == 1)
        def _():
            pltpu.make_async_copy(
                rows.at[pl.ds(off, 16), :], out_hbm.at[pos16], sem_fl1).start()
            pltpu.make_async_copy(
                out_hbm.at[pl.ds(0, 16), :],
                rows.at[pl.ds(0, 16), :], sem_fl1).wait()

        @pl.when((out_fl >= 1) & (fpar == 1))
        def _():
            pltpu.make_async_copy(
                out_hbm.at[pl.ds(0, 16), :],
                rows.at[pl.ds(0, 16), :], sem_fl0).wait()

        @pl.when((out_fl >= 1) & (fpar == 0))
        def _():
            pltpu.make_async_copy(
                out_hbm.at[pl.ds(0, 16), :],
                rows.at[pl.ds(0, 16), :], sem_fl1).wait()

    return sweep_k(dest, table_t, tail_t)


def _uw_body(idx_ref, tab_ref, w_ref, uw_ref):
    oh = jnp.zeros((_BM, _TAB_PAD), jnp.float32)
    iota = lax.broadcasted_iota(jnp.int32, (_BM, _TAB_PAD), 1)
    for k in range(6):
        oh = oh + (idx_ref[:, k : k + 1] == iota).astype(jnp.float32)
    user = jnp.dot(oh, tab_ref[...], preferred_element_type=jnp.float32,
                   precision=lax.Precision.DEFAULT)
    uw_ref[:, :_F] = user * w_ref[0:1, :]
    uw_ref[:, _F:] = jnp.zeros((_BM, _F), jnp.float32)


def _sc_reduce(cols, uw, bias):
    B = cols.shape[0] - 16
    try:
        info = plsc.get_sparse_core_info()
        nc, ns = info.num_cores, info.num_subcores
    except Exception:
        nc, ns = 2, 16
    nw = nc * ns
    bpw = B // nw
    ngr = bpw // 16
    mesh = plsc.VectorSubcoreMesh(core_axis_name="c", subcore_axis_name="s")

    @functools.partial(
        pl.kernel,
        mesh=mesh,
        compiler_params=pltpu.CompilerParams(needs_layout_passes=False),
        out_type=jax.ShapeDtypeStruct((B,), jnp.float32),
        scratch_types=[
            pltpu.VMEM((32, 128), jnp.float32),
            pltpu.VMEM((32, 128), jnp.float32),
            pltpu.VMEM((bpw,), jnp.float32),
            pltpu.VMEM((16,), jnp.float32),
            pltpu.SemaphoreType.DMA,
            pltpu.SemaphoreType.DMA,
        ],
    )
    def reduce_k(cols_hbm, uw_hbm, bias_hbm, out_hbm,
                 colsb, uwb, out_v, bias_v, sem0, sem1):
        wid = lax.axis_index("s") * nc + lax.axis_index("c")
        base = wid * bpw
        iota16 = lax.broadcasted_iota(jnp.int32, (16,), 0)
        pltpu.sync_copy(bias_hbm, bias_v)
        bias_vec = bias_v[...]
        sems = (sem0, sem1)

        def fire(g):
            h = lax.rem(g, 2)
            r0 = pl.multiple_of(base + g * 16, 16)
            dh = pl.multiple_of(h * 16, 16)
            for hh in range(2):
                @pl.when(h == hh)
                def _(hh=hh):
                    pltpu.make_async_copy(
                        cols_hbm.at[pl.ds(r0, 16), :],
                        colsb.at[pl.ds(dh, 16), :], sems[hh]).start()
                    pltpu.make_async_copy(
                        uw_hbm.at[pl.ds(r0, 16), :],
                        uwb.at[pl.ds(dh, 16), :], sems[hh]).start()

        def wait_grp(g):
            h = lax.rem(g, 2)
            for hh in range(2):
                @pl.when(h == hh)
                def _(hh=hh):
                    pltpu.make_async_copy(
                        cols_hbm.at[pl.ds(0, 16), :],
                        colsb.at[pl.ds(0, 16), :], sems[hh]).wait()
                    pltpu.make_async_copy(
                        uw_hbm.at[pl.ds(0, 16), :],
                        uwb.at[pl.ds(0, 16), :], sems[hh]).wait()

        fire(0)

        def gloop(g, _):
            @pl.when(g + 1 < ngr)
            def _():
                fire(g + 1)
            wait_grp(g)
            h = lax.rem(g, 2)
            rowv = h * 16 + iota16
            acc = jnp.zeros((16,), jnp.float32)
            for f in range(_F):
                fv = jnp.full((16,), f, jnp.int32)
                cf = plsc.load_gather(colsb, [rowv, fv])
                uf = plsc.load_gather(uwb, [rowv, fv])
                acc = acc + cf * uf
            acc = acc + bias_vec
            res = jnp.where(acc >= 0, acc, 0.01 * acc)
            g16 = pl.multiple_of(g * 16, 16)
            out_v[pl.ds(g16, 16)] = res
            return 0
        lax.fori_loop(0, ngr, gloop, 0)
        pltpu.sync_copy(out_v, out_hbm.at[pl.ds(base, bpw)])

    return reduce_k(cols, uw, jnp.broadcast_to(bias, (16,)))


def kernel(dayofweek, time, sex, age, month, day, destination,
           emb_dayofweek, emb_time, emb_sex, emb_age, emb_month, emb_day,
           item_table, W, b):
    B = destination.shape[0]
    dest = destination.astype(jnp.int32)

    table_t = item_table.T
    ntf = table_t.shape[1] // 256
    tail_t = table_t[:, ntf * 256 :]
    cols = _sc_sweep(table_t, tail_t, dest)

    tab = jnp.concatenate(
        [emb_dayofweek, emb_time, emb_sex, emb_age, emb_month, emb_day], axis=0)
    tab = jnp.pad(tab, ((0, _TAB_PAD - tab.shape[0]), (0, 0)))
    offs = (0, 7, 31, 33, 133, 145)
    feats = (dayofweek, time, sex, age, month, day)
    idx_cols = [f.astype(jnp.int32) + o for f, o in zip(feats, offs)]
    idx_cols += [jnp.zeros((B,), jnp.int32)] * 2
    idx_all = jnp.stack(idx_cols, axis=1)
    w_pad = jnp.pad(W, ((0, 7), (0, 0)))
    b2 = b.reshape(1, 1)

    nblk = B // _BM
    uw = pl.pallas_call(
        _uw_body,
        grid=(nblk,),
        in_specs=[
            pl.BlockSpec((_BM, 8), lambda i: (i, 0)),
            pl.BlockSpec((_TAB_PAD, _F), lambda i: (0, 0)),
            pl.BlockSpec((8, _F), lambda i: (0, 0)),
        ],
        out_specs=pl.BlockSpec((_BM, 2 * _F), lambda i: (i, 0)),
        out_shape=jax.ShapeDtypeStruct((B + 16, 2 * _F), jnp.float32),
    )(idx_all, tab, w_pad)
    return _sc_reduce(cols, uw, b)
